# baseline (speedup 1.0000x reference)
import functools

import jax
import jax.numpy as jnp
from jax import lax
from jax.experimental import pallas as pl
from jax.experimental.pallas import tpu as pltpu
from jax.experimental.pallas import tpu_sc as plsc

_BT = 1920
_SC_T = 512
_NW = 32
_TPW = _SC_T // _NW


def _gate_block(x, wt, bias):
    logits = jnp.dot(x, wt, preferred_element_type=jnp.float32) + bias
    m = jnp.max(logits, axis=1, keepdims=True)
    e = jnp.exp(logits - m)
    s = jnp.sum(e, axis=1, keepdims=True)
    sm = e / s
    v = 1.0 / s
    lane = jax.lax.broadcasted_iota(jnp.int32, sm.shape, 1).astype(jnp.float32)
    idxf = jnp.min(jnp.where(sm >= v, lane, float(sm.shape[1])),
                   axis=1, keepdims=True)
    return idxf.astype(jnp.int32), v


def _tc_body(x_ref, wt_ref, bias_ref, idx_ref, score_ref):
    i, v = _gate_block(x_ref[...], wt_ref[...], bias_ref[...])
    idx_ref[...] = i
    score_ref[...] = v


def _tc_gate(inp, tc_t, wt, bias):
    D = inp.shape[1]
    E = wt.shape[1]
    grid = (tc_t // _BT,)
    return pl.pallas_call(
        _tc_body,
        grid=grid,
        in_specs=[
            pl.BlockSpec((_BT, D), lambda i: (i, 0)),
            pl.BlockSpec((D, E), lambda i: (0, 0)),
            pl.BlockSpec((1, E), lambda i: (0, 0)),
        ],
        out_specs=[
            pl.BlockSpec((_BT, 1), lambda i: (i, 0)),
            pl.BlockSpec((_BT, 1), lambda i: (i, 0)),
        ],
        out_shape=[
            jax.ShapeDtypeStruct((tc_t, 1), jnp.int32),
            jax.ShapeDtypeStruct((tc_t, 1), jnp.float32),
        ],
    )(inp, wt, bias)


def _sc_body(inp_hbm, wt_hbm, b_hbm, idx_hbm, score_hbm, x_v, wt_v, b_v,
             res_v):
    wid = lax.axis_index("s") * 2 + lax.axis_index("c")
    base = (inp_hbm.shape[0] - _SC_T) + wid * _TPW
    obase = wid * _TPW
    pltpu.sync_copy(wt_hbm, wt_v)
    pltpu.sync_copy(b_hbm, b_v)
    pltpu.sync_copy(inp_hbm.at[pl.ds(base, _TPW)], x_v)
    lanes = lax.iota(jnp.int32, 16)

    def shuffle(x, sh):
        idxv = jnp.bitwise_xor(lanes, sh)
        return lax.gather(
            x, idxv[:, None],
            lax.GatherDimensionNumbers(
                offset_dims=(), collapsed_slice_dims=(0,),
                start_index_map=(0,)),
            slice_sizes=(1,),
            mode=lax.GatherScatterMode.PROMISE_IN_BOUNDS)

    def allreduce(x, op):
        for sh in (1, 2, 4, 8):
            x = op(x, shuffle(x, sh))
        return x
    res_idx = jnp.zeros((16,), jnp.float32)
    res_score = jnp.zeros((16,), jnp.float32)
    bias = b_v[...]
    for t in range(_TPW):
        def dstep(i, acc):
            d0 = i * 16
            xchunk = x_v[t, pl.ds(d0, 16)]
            for j in range(16):
                acc = acc + xchunk[j] * wt_v[pl.ds((d0 + j) * 16, 16)]
            return acc
        acc = lax.fori_loop(0, 64, dstep, jnp.zeros((16,), jnp.float32))
        logits = acc + bias
        m = allreduce(logits, jnp.maximum)
        e = jnp.exp(logits - m)
        s = allreduce(e, jnp.add)
        sm = e / s
        v = 1.0 / s
        lanef = lanes.astype(jnp.float32)
        first = allreduce(jnp.where(sm >= v, lanef, 16.0), jnp.minimum)
        tmask = lanes == t
        res_idx = jnp.where(tmask, first, res_idx)
        res_score = jnp.where(tmask, v, res_score)
    res_v[0, :] = res_idx
    res_v[1, :] = res_score
    pltpu.sync_copy(res_v.at[0], idx_hbm.at[pl.ds(obase, _TPW)])
    pltpu.sync_copy(res_v.at[1], score_hbm.at[pl.ds(obase, _TPW)])


def _sc_gate(inp, wt_flat, b):
    k = pl.kernel(
        _sc_body,
        out_type=[
            jax.ShapeDtypeStruct((_SC_T,), jnp.float32),
            jax.ShapeDtypeStruct((_SC_T,), jnp.float32),
        ],
        mesh=plsc.VectorSubcoreMesh(core_axis_name="c", subcore_axis_name="s"),
        scratch_types=[
            pltpu.VMEM((_TPW, 1024), jnp.float32),
            pltpu.VMEM((1024 * 16,), jnp.float32),
            pltpu.VMEM((16,), jnp.float32),
            pltpu.VMEM((2, 16), jnp.float32),
        ],
    )
    idxf, score = k(inp, wt_flat, b)
    return idxf.astype(jnp.int32), score


def kernel(inp, W, b):
    T, D = inp.shape
    E = W.shape[0]
    wt = W.T
    bias = b.reshape(1, E)
    tc_t = T - _SC_T
    idx_tc, score_tc = _tc_gate(inp, tc_t, wt, bias)
    idx_sc, score_sc = _sc_gate(inp, wt.reshape(-1), b)
    idx = jnp.concatenate([idx_tc, idx_sc.reshape(_SC_T, 1)], axis=0)
    score = jnp.concatenate([score_tc, score_sc.reshape(_SC_T, 1)], axis=0)
    return (idx.astype(jnp.int64), score)


# speedup vs baseline: 2.4319x; 2.4319x over previous
import jax
import jax.numpy as jnp
from jax.experimental import pallas as pl

_BT = 2048


def _gate_body(x_ref, wt_ref, bias_ref, idx_ref, score_ref):
    x = x_ref[...]
    logits = jnp.dot(x, wt_ref[...], preferred_element_type=jnp.float32)
    logits = logits + bias_ref[...]
    m = jnp.max(logits, axis=1, keepdims=True)
    e = jnp.exp(logits - m)
    s = jnp.sum(e, axis=1, keepdims=True)
    sm = e / s
    v = 1.0 / s
    lane = jax.lax.broadcasted_iota(jnp.int32, sm.shape, 1).astype(jnp.float32)
    idxf = jnp.min(jnp.where(sm >= v, lane, float(sm.shape[1])),
                   axis=1, keepdims=True)
    idx_ref[...] = idxf.astype(jnp.int32)
    score_ref[...] = v


def kernel(inp, W, b):
    T, D = inp.shape
    E = W.shape[0]
    wt = W.T
    bias = b.reshape(1, E)
    grid = (T // _BT,)
    idx, score = pl.pallas_call(
        _gate_body,
        grid=grid,
        in_specs=[
            pl.BlockSpec((_BT, D), lambda i: (i, 0)),
            pl.BlockSpec((D, E), lambda i: (0, 0)),
            pl.BlockSpec((1, E), lambda i: (0, 0)),
        ],
        out_specs=[
            pl.BlockSpec((_BT, 1), lambda i: (i, 0)),
            pl.BlockSpec((_BT, 1), lambda i: (i, 0)),
        ],
        out_shape=[
            jax.ShapeDtypeStruct((T, 1), jnp.int32),
            jax.ShapeDtypeStruct((T, 1), jnp.float32),
        ],
    )(inp, wt, bias)
    return (idx.astype(jnp.int64), score)
